# baseline (speedup 1.0000x reference)
import functools

import jax
import jax.numpy as jnp
from jax import lax
from jax.experimental import pallas as pl
from jax.experimental.pallas import tpu as pltpu
from jax.experimental.pallas import tpu_sc as plsc


_SC_CHUNKS = 1


def _sc_gather(table, idx_flat):
    info = plsc.get_sparse_core_info()
    num_workers = info.num_cores * info.num_subcores
    b = idx_flat.shape[0]
    d = table.shape[1]
    b_per_w = b // num_workers
    rows_per_chunk = b_per_w // _SC_CHUNKS
    mesh = plsc.VectorSubcoreMesh(core_axis_name="c", subcore_axis_name="s")

    @functools.partial(
        pl.kernel,
        mesh=mesh,
        out_type=jax.ShapeDtypeStruct((b, d), jnp.float32),
        scratch_types=[
            pltpu.VMEM((b_per_w,), jnp.int32),
        ]
        + [pltpu.VMEM((rows_per_chunk, d), jnp.float32)] * _SC_CHUNKS
        + [pltpu.SemaphoreType.DMA] * (2 * _SC_CHUNKS),
    )
    def gather_k(table_hbm, idx_hbm, out_hbm, idx_v, *bufs_and_sems):
        bufs = bufs_and_sems[:_SC_CHUNKS]
        gsems = bufs_and_sems[_SC_CHUNKS : 2 * _SC_CHUNKS]
        osems = bufs_and_sems[2 * _SC_CHUNKS :]
        wid = lax.axis_index("s") * info.num_cores + lax.axis_index("c")
        base = wid * b_per_w
        pltpu.sync_copy(idx_hbm.at[pl.ds(base, b_per_w)], idx_v)
        gathers = [
            pltpu.async_copy(
                table_hbm.at[idx_v.at[pl.ds(c * rows_per_chunk, rows_per_chunk)]],
                bufs[c],
                gsems[c],
            )
            for c in range(_SC_CHUNKS)
        ]
        scatters = []
        for c in range(_SC_CHUNKS):
            gathers[c].wait()
            scatters.append(
                pltpu.async_copy(
                    bufs[c],
                    out_hbm.at[pl.ds(base + c * rows_per_chunk, rows_per_chunk)],
                    osems[c],
                )
            )
        for s in scatters:
            s.wait()

    return gather_k(table, idx_flat)


_BATCHES_PER_STEP = 8


def _tc_body(tok_ref, tok_tail_ref, vc_ref, out_ref):
    for j in range(_BATCHES_PER_STEP):
        t = tok_ref[j]
        tl = tok_tail_ref[j, 0:1]
        vc = vc_ref[j]
        k = vc.shape[0]
        sims = lax.dot_general(
            vc, t, (((1,), (1,)), ((), ())), preferred_element_type=jnp.float32
        )
        sims_l = lax.dot_general(
            vc, tl, (((1,), (1,)), ((), ())), preferred_element_type=jnp.float32
        )
        s2 = jnp.concatenate([sims, -sims], axis=0)
        s2l = jnp.concatenate([sims_l, -sims_l], axis=0)
        col = lax.broadcasted_iota(jnp.int32, s2.shape, 1)
        s2 = jnp.where(col == 0, -jnp.inf, s2)
        e = jnp.exp(s2)
        el = jnp.exp(s2l)
        w = lax.dot_general(
            e, t, (((1,), (0,)), ((), ())), preferred_element_type=jnp.float32
        ) + lax.dot_general(
            el, tl, (((1,), (0,)), ((), ())), preferred_element_type=jnp.float32
        )
        w = w * lax.rsqrt(jnp.maximum(jnp.sum(w * w, axis=-1, keepdims=True), 1e-30))
        out_ref[0, j] = w[:k]
        out_ref[1, j] = w[k:]


def kernel(tokens, topk_indices, visual_concepts):
    b, n1, d = tokens.shape
    k = topk_indices.shape[1]
    idx_flat = topk_indices.reshape(-1).astype(jnp.int32)
    vc = _sc_gather(visual_concepts, idx_flat).reshape(b, k, d)
    g = _BATCHES_PER_STEP
    out = pl.pallas_call(
        _tc_body,
        grid=(b // g,),
        in_specs=[
            pl.BlockSpec((g, n1 - 1, d), lambda i: (i, 0, 0)),
            pl.BlockSpec((g, 8, d), lambda i: (i, (n1 - 1) // 8, 0)),
        pl.BlockSpec((g, k, d), lambda i: (i, 0, 0)),
        ],
        out_specs=pl.BlockSpec((2, g, k, d), lambda i: (0, i, 0, 0)),
        out_shape=jax.ShapeDtypeStruct((2, b, k, d), jnp.float32),
        compiler_params=pltpu.CompilerParams(
            dimension_semantics=("parallel",),
        ),
    )(tokens, tokens, vc)
    return out


# speedup vs baseline: 1.0026x; 1.0026x over previous
import functools

import jax
import jax.numpy as jnp
from jax import lax
from jax.experimental import pallas as pl
from jax.experimental.pallas import tpu as pltpu
from jax.experimental.pallas import tpu_sc as plsc


_SC_CHUNKS = 1


def _sc_gather(table, idx_flat):
    info = plsc.get_sparse_core_info()
    num_workers = info.num_cores * info.num_subcores
    b = idx_flat.shape[0]
    d = table.shape[1]
    b_per_w = b // num_workers
    mesh = plsc.VectorSubcoreMesh(core_axis_name="c", subcore_axis_name="s")

    @functools.partial(
        pl.kernel,
        mesh=mesh,
        out_type=jax.ShapeDtypeStruct((b, d), jnp.float32),
        scratch_types=[
            pltpu.VMEM((b_per_w,), jnp.int32),
            pltpu.VMEM((b_per_w, d), jnp.float32),
            pltpu.SemaphoreType.DMA,
        ],
    )
    def gather_k(table_hbm, idx_hbm, out_hbm, idx_v, rows_v, sem):
        wid = lax.axis_index("s") * info.num_cores + lax.axis_index("c")
        base = wid * b_per_w
        pltpu.sync_copy(idx_hbm.at[pl.ds(base, b_per_w)], idx_v)
        pltpu.async_copy(table_hbm.at[idx_v], rows_v, sem).wait()
        pltpu.sync_copy(rows_v, out_hbm.at[pl.ds(base, b_per_w)])

    return gather_k(table, idx_flat)


_BATCHES_PER_STEP = 8


def _tc_body(tok_ref, tok_tail_ref, vc_ref, out_ref):
    for j in range(_BATCHES_PER_STEP):
        t = tok_ref[j]
        tl = tok_tail_ref[j, 0:1]
        vc = vc_ref[j]
        k = vc.shape[0]
        sims = lax.dot_general(
            vc, t, (((1,), (1,)), ((), ())), preferred_element_type=jnp.float32
        )
        sims_l = lax.dot_general(
            vc, tl, (((1,), (1,)), ((), ())), preferred_element_type=jnp.float32
        )
        s2 = jnp.concatenate([sims, -sims], axis=0)
        s2l = jnp.concatenate([sims_l, -sims_l], axis=0)
        col = lax.broadcasted_iota(jnp.int32, s2.shape, 1)
        s2 = jnp.where(col == 0, -jnp.inf, s2)
        e = jnp.exp(s2)
        el = jnp.exp(s2l)
        w = lax.dot_general(
            e, t, (((1,), (0,)), ((), ())), preferred_element_type=jnp.float32
        ) + lax.dot_general(
            el, tl, (((1,), (0,)), ((), ())), preferred_element_type=jnp.float32
        )
        w = w * lax.rsqrt(jnp.maximum(jnp.sum(w * w, axis=-1, keepdims=True), 1e-30))
        out_ref[0, j] = w[:k]
        out_ref[1, j] = w[k:]


def kernel(tokens, topk_indices, visual_concepts):
    b, n1, d = tokens.shape
    k = topk_indices.shape[1]
    idx_flat = topk_indices.reshape(-1).astype(jnp.int32)
    vc = _sc_gather(visual_concepts, idx_flat).reshape(b, k, d)
    g = _BATCHES_PER_STEP
    out = pl.pallas_call(
        _tc_body,
        grid=(b // g,),
        in_specs=[
            pl.BlockSpec((g, n1 - 1, d), lambda i: (i, 0, 0)),
            pl.BlockSpec((g, 8, d), lambda i: (i, (n1 - 1) // 8, 0)),
        pl.BlockSpec((g, k, d), lambda i: (i, 0, 0)),
        ],
        out_specs=pl.BlockSpec((2, g, k, d), lambda i: (0, i, 0, 0)),
        out_shape=jax.ShapeDtypeStruct((2, b, k, d), jnp.float32),
        compiler_params=pltpu.CompilerParams(
            dimension_semantics=("parallel",),
        ),
    )(tokens, tokens, vc)
    return out


# speedup vs baseline: 1.0158x; 1.0132x over previous
import functools

import jax
import jax.numpy as jnp
from jax import lax
from jax.experimental import pallas as pl
from jax.experimental.pallas import tpu as pltpu
from jax.experimental.pallas import tpu_sc as plsc


_SC_CHUNKS = 1


def _sc_gather(table, idx_flat):
    info = plsc.get_sparse_core_info()
    num_workers = info.num_cores * info.num_subcores
    b = idx_flat.shape[0]
    d = table.shape[1]
    b_per_w = b // num_workers
    mesh = plsc.VectorSubcoreMesh(core_axis_name="c", subcore_axis_name="s")

    @functools.partial(
        pl.kernel,
        mesh=mesh,
        out_type=jax.ShapeDtypeStruct((b, d), jnp.float32),
        scratch_types=[
            pltpu.VMEM((b_per_w,), jnp.int32),
            pltpu.VMEM((b_per_w, d), jnp.float32),
            pltpu.SemaphoreType.DMA,
        ],
    )
    def gather_k(table_hbm, idx_hbm, out_hbm, idx_v, rows_v, sem):
        wid = lax.axis_index("s") * info.num_cores + lax.axis_index("c")
        base = wid * b_per_w
        pltpu.sync_copy(idx_hbm.at[pl.ds(base, b_per_w)], idx_v)
        pltpu.async_copy(table_hbm.at[idx_v], rows_v, sem).wait()
        pltpu.sync_copy(rows_v, out_hbm.at[pl.ds(base, b_per_w)])

    return gather_k(table, idx_flat)


_BATCHES_PER_STEP = 16


def _tc_body(tok_ref, tok_tail_ref, vc_ref, out_ref):
    for j in range(_BATCHES_PER_STEP):
        t = tok_ref[j]
        tl = tok_tail_ref[j, 0:1]
        vc = vc_ref[j]
        k = vc.shape[0]
        sims = lax.dot_general(
            vc, t, (((1,), (1,)), ((), ())), preferred_element_type=jnp.float32
        )
        sims_l = lax.dot_general(
            vc, tl, (((1,), (1,)), ((), ())), preferred_element_type=jnp.float32
        )
        s2 = jnp.concatenate([sims, -sims], axis=0)
        s2l = jnp.concatenate([sims_l, -sims_l], axis=0)
        col = lax.broadcasted_iota(jnp.int32, s2.shape, 1)
        s2 = jnp.where(col == 0, -jnp.inf, s2)
        e = jnp.exp(s2)
        el = jnp.exp(s2l)
        w = lax.dot_general(
            e, t, (((1,), (0,)), ((), ())), preferred_element_type=jnp.float32
        ) + lax.dot_general(
            el, tl, (((1,), (0,)), ((), ())), preferred_element_type=jnp.float32
        )
        w = w * lax.rsqrt(jnp.maximum(jnp.sum(w * w, axis=-1, keepdims=True), 1e-30))
        out_ref[0, j] = w[:k]
        out_ref[1, j] = w[k:]


def kernel(tokens, topk_indices, visual_concepts):
    b, n1, d = tokens.shape
    k = topk_indices.shape[1]
    idx_flat = topk_indices.reshape(-1).astype(jnp.int32)
    vc = _sc_gather(visual_concepts, idx_flat).reshape(b, k, d)
    g = _BATCHES_PER_STEP
    out = pl.pallas_call(
        _tc_body,
        grid=(b // g,),
        in_specs=[
            pl.BlockSpec((g, n1 - 1, d), lambda i: (i, 0, 0)),
            pl.BlockSpec((g, 8, d), lambda i: (i, (n1 - 1) // 8, 0)),
        pl.BlockSpec((g, k, d), lambda i: (i, 0, 0)),
        ],
        out_specs=pl.BlockSpec((2, g, k, d), lambda i: (0, i, 0, 0)),
        out_shape=jax.ShapeDtypeStruct((2, b, k, d), jnp.float32),
        compiler_params=pltpu.CompilerParams(
            dimension_semantics=("parallel",),
        ),
    )(tokens, tokens, vc)
    return out


# speedup vs baseline: 1.0193x; 1.0034x over previous
import functools

import jax
import jax.numpy as jnp
from jax import lax
from jax.experimental import pallas as pl
from jax.experimental.pallas import tpu as pltpu
from jax.experimental.pallas import tpu_sc as plsc


def _sc_gather(table, idx_flat):
    info = plsc.get_sparse_core_info()
    num_workers = info.num_cores * info.num_subcores
    b = idx_flat.shape[0]
    d = table.shape[1]
    b_per_w = b // num_workers
    mesh = plsc.VectorSubcoreMesh(core_axis_name="c", subcore_axis_name="s")

    @functools.partial(
        pl.kernel,
        mesh=mesh,
        out_type=jax.ShapeDtypeStruct((b, d), jnp.float32),
        scratch_types=[
            pltpu.VMEM((b_per_w,), jnp.int32),
            pltpu.VMEM((b_per_w, d), jnp.float32),
            pltpu.SemaphoreType.DMA,
        ],
    )
    def gather_k(table_hbm, idx_hbm, out_hbm, idx_v, rows_v, sem):
        wid = lax.axis_index("s") * info.num_cores + lax.axis_index("c")
        base = wid * b_per_w
        pltpu.sync_copy(idx_hbm.at[pl.ds(base, b_per_w)], idx_v)
        pltpu.async_copy(table_hbm.at[idx_v], rows_v, sem).wait()
        pltpu.sync_copy(rows_v, out_hbm.at[pl.ds(base, b_per_w)])

    return gather_k(table, idx_flat)


_BATCHES_PER_STEP = 16


def _tc_body(tok_ref, tok_tail_ref, vc_ref, out_ref):
    for j in range(_BATCHES_PER_STEP):
        t = tok_ref[j]
        tl = tok_tail_ref[j, 0:1]
        vc = vc_ref[j]
        k = vc.shape[0]
        sims = lax.dot_general(
            vc, t, (((1,), (1,)), ((), ())), preferred_element_type=jnp.float32
        )
        sims_l = lax.dot_general(
            vc, tl, (((1,), (1,)), ((), ())), preferred_element_type=jnp.float32
        )
        s2 = jnp.concatenate([sims, -sims], axis=0)
        s2l = jnp.concatenate([sims_l, -sims_l], axis=0)
        col = lax.broadcasted_iota(jnp.int32, s2.shape, 1)
        s2 = jnp.where(col == 0, -jnp.inf, s2)
        e = jnp.exp(s2)
        el = jnp.exp(s2l)
        w = lax.dot_general(
            e, t, (((1,), (0,)), ((), ())), preferred_element_type=jnp.float32
        ) + lax.dot_general(
            el, tl, (((1,), (0,)), ((), ())), preferred_element_type=jnp.float32
        )
        w = w * lax.rsqrt(jnp.maximum(jnp.sum(w * w, axis=-1, keepdims=True), 1e-30))
        out_ref[0, j] = w[:k]
        out_ref[1, j] = w[k:]


def kernel(tokens, topk_indices, visual_concepts):
    b, n1, d = tokens.shape
    k = topk_indices.shape[1]
    idx_flat = topk_indices.reshape(-1).astype(jnp.int32)
    vc = _sc_gather(visual_concepts, idx_flat).reshape(b, k, d)
    g = _BATCHES_PER_STEP
    out = pl.pallas_call(
        _tc_body,
        grid=(b // g,),
        in_specs=[
            pl.BlockSpec((g, n1 - 1, d), lambda i: (i, 0, 0)),
            pl.BlockSpec((g, 8, d), lambda i: (i, (n1 - 1) // 8, 0)),
            pl.BlockSpec((g, k, d), lambda i: (i, 0, 0)),
        ],
        out_specs=pl.BlockSpec((2, g, k, d), lambda i: (0, i, 0, 0)),
        out_shape=jax.ShapeDtypeStruct((2, b, k, d), jnp.float32),
        compiler_params=pltpu.CompilerParams(
            dimension_semantics=("parallel",),
        ),
    )(tokens, tokens, vc)
    return out
